# Initial kernel scaffold; baseline (speedup 1.0000x reference)
import jax
import jax.numpy as jnp
from jax.experimental import pallas as pl


def kernel(node_features, edge_features, edge_index, enc_n_W1, enc_n_b1, enc_n_W2, enc_n_b2, enc_n_g, enc_n_be, enc_e_W1, enc_e_b1, enc_e_W2, enc_e_b2, enc_e_g, enc_e_be, pe_W1, pe_b1, pe_W2, pe_b2, pe_g, pe_be, pn_W1, pn_b1, pn_W2, pn_b2, pn_g, pn_be, dec_W1, dec_b1, dec_W2, dec_b2):
    raise NotImplementedError("write your pallas kernel here")


# speedup vs baseline: 1.9931x; 1.9931x over previous
import functools

import jax
import jax.numpy as jnp
from jax import lax
from jax.experimental import pallas as pl
from jax.experimental.pallas import tpu as pltpu
from jax.experimental.pallas import tpu_sc as plsc

_N = 10000
_E = 160000
_H = 64
_W = 128
_P = 8

_NC = 2
_NS = 16
_NW = _NC * _NS
_CH = 128
_NCH = 40
_EPW = _CH * _NCH
_EP = _EPW * _NW
_ACC = 10240
_RPT = _ACC // _NS

_EB = 4096
_NB = 1000

_sc_mesh = plsc.VectorSubcoreMesh(core_axis_name="c", subcore_axis_name="s")


@functools.partial(
    pl.kernel,
    mesh=_sc_mesh,
    out_type=[
        jax.ShapeDtypeStruct((_EP, _W), jnp.float32),
        jax.ShapeDtypeStruct((_EP, _W), jnp.float32),
    ],
    scratch_types=[
        pltpu.VMEM((_NCH, _CH), jnp.int32),
        pltpu.VMEM((_NCH, _CH), jnp.int32),
        pltpu.VMEM((_CH, _W), jnp.float32),
        pltpu.VMEM((_CH, _W), jnp.float32),
        pltpu.VMEM((_CH, _W), jnp.float32),
        pltpu.VMEM((_CH, _W), jnp.float32),
        pltpu.SemaphoreType.DMA,
        pltpu.SemaphoreType.DMA,
        pltpu.SemaphoreType.DMA,
        pltpu.SemaphoreType.DMA,
    ],
)
def _sc_gather2(w_hbm, src_hbm, dst_hbm, ga_hbm, gb_hbm,
                idxs, idxd, bs0, bs1, bd0, bd1, ss0, ss1, sd0, sd1):
    wid = lax.axis_index("s") * _NC + lax.axis_index("c")
    crow = wid * _NCH
    erow = wid * _EPW
    pltpu.sync_copy(src_hbm.at[pl.ds(crow, _NCH)], idxs)
    pltpu.sync_copy(dst_hbm.at[pl.ds(crow, _NCH)], idxd)

    def start(j):
        p = j % 2
        cs = pltpu.async_copy(w_hbm.at[idxs.at[j]], (bs0, bs1)[p], (ss0, ss1)[p])
        cd = pltpu.async_copy(w_hbm.at[idxd.at[j]], (bd0, bd1)[p], (sd0, sd1)[p])
        return cs, cd

    pend = [start(0), start(1)]
    for j in range(_NCH):
        p = j % 2
        cs, cd = pend[p]
        cs.wait()
        cd.wait()
        pltpu.sync_copy((bs0, bs1)[p], ga_hbm.at[pl.ds(erow + j * _CH, _CH)])
        pltpu.sync_copy((bd0, bd1)[p], gb_hbm.at[pl.ds(erow + j * _CH, _CH)])
        if j + 2 < _NCH:
            pend[p] = start(j + 2)


@functools.partial(
    pl.kernel,
    mesh=_sc_mesh,
    out_type=jax.ShapeDtypeStruct((_NC * _ACC, _W), jnp.float32),
    scratch_types=[
        pltpu.VMEM((_NCH, _CH), jnp.int32),
        pltpu.VMEM((_CH, _W), jnp.float32),
        pltpu.VMEM((_CH, _W), jnp.float32),
        pltpu.VMEM_SHARED((_ACC, _W), jnp.float32),
        pltpu.SemaphoreType.DMA,
        pltpu.SemaphoreType.DMA,
    ],
)
def _sc_scatter(vals_hbm, dsti_hbm, zero_hbm, out_hbm,
                idx_v, buf0, buf1, acc, sem0, sem1):
    cid = lax.axis_index("c")
    sid = lax.axis_index("s")
    wid = sid * _NC + cid

    pltpu.sync_copy(zero_hbm, buf0)
    for k in range(_RPT // _CH):
        pltpu.sync_copy(buf0, acc.at[pl.ds(sid * _RPT + k * _CH, _CH)])

    pltpu.sync_copy(dsti_hbm.at[pl.ds(wid * _NCH, _NCH)], idx_v)

    def start(j):
        p = j % 2
        return pltpu.async_copy(
            vals_hbm.at[pl.ds(wid * _EPW + j * _CH, _CH)],
            (buf0, buf1)[p], (sem0, sem1)[p])

    plsc.subcore_barrier()
    pend = [start(0), start(1)]
    for j in range(_NCH):
        p = j % 2
        pend[p].wait()
        pltpu.sync_copy((buf0, buf1)[p], acc.at[idx_v.at[j]], add=True)
        if j + 2 < _NCH:
            pend[p] = start(j + 2)
    plsc.subcore_barrier()

    pltpu.sync_copy(acc.at[pl.ds(sid * _RPT, _RPT)],
                    out_hbm.at[pl.ds(cid * _ACC + sid * _RPT, _RPT)])


def _ln(h, g, be):
    mu = jnp.mean(h, axis=-1, keepdims=True)
    var = jnp.mean((h - mu) ** 2, axis=-1, keepdims=True)
    return g * (h - mu) * lax.rsqrt(var + 1e-5) + be


def _dot(a, b):
    return jnp.dot(a, b, preferred_element_type=jnp.float32)


def _full(shape):
    return pl.BlockSpec(shape, lambda i: tuple(0 for _ in shape))


def _pack_zeros(e):
    return jnp.concatenate([e, jnp.zeros_like(e)], axis=1)


def _enc_node_body(nf, W1, b1, W2, b2, g, be, Ws, Wd, x_out, w_out):
    h = jnp.maximum(_dot(nf[...], W1[...]) + b1[...], 0.0)
    h = _dot(h, W2[...]) + b2[...]
    x = _ln(h, g[...], be[...])
    x_out[...] = x
    w_out[...] = jnp.concatenate([_dot(x, Ws[...]), _dot(x, Wd[...])], axis=1)


def _enc_edge_body(ef, W1, b1, W2, b2, g, be, e_out):
    h = jnp.maximum(_dot(ef[...], W1[...]) + b1[...], 0.0)
    h = _dot(h, W2[...]) + b2[...]
    e_out[...] = _pack_zeros(_ln(h, g[...], be[...]))


def _edge_step_body(e2, ga, gb, W1e, b1, W2, b2, g, be, e_out):
    el = e2[...][:, :_H]
    s = ga[...][:, :_H] + gb[...][:, _H:]
    h = jnp.maximum(_dot(el, W1e[...]) + s + b1[...], 0.0)
    h = _dot(h, W2[...]) + b2[...]
    e_out[...] = _pack_zeros(_ln(h, g[...], be[...]) + el)


def _node_step_body(x, a0, a1, W1x, W1a, b1, W2, b2, g, be, Ws, Wd,
                    x_out, w_out):
    agg = a0[...][:, :_H] + a1[...][:, :_H]
    h = jnp.maximum(_dot(x[...], W1x[...]) + _dot(agg, W1a[...]) + b1[...], 0.0)
    h = _dot(h, W2[...]) + b2[...]
    xn = _ln(h, g[...], be[...]) + x[...]
    x_out[...] = xn
    w_out[...] = jnp.concatenate([_dot(xn, Ws[...]), _dot(xn, Wd[...])], axis=1)


def _dec_body(x, W1, b1, W2, b2, y_out):
    h = jnp.maximum(_dot(x[...], W1[...]) + b1[...], 0.0)
    y_out[...] = _dot(h, W2[...]) + b2[...]


def _node_grid_call(body, arrs, weights):
    blk = lambda w: pl.BlockSpec((_NB, w), lambda i: (i, 0))
    f32 = jnp.float32
    return pl.pallas_call(
        body,
        grid=(_N // _NB,),
        in_specs=[blk(a.shape[1]) for a in arrs] + [_full(w.shape) for w in weights],
        out_specs=[blk(_H), blk(_W)],
        out_shape=[jax.ShapeDtypeStruct((_N, _H), f32),
                   jax.ShapeDtypeStruct((_N, _W), f32)],
    )(*arrs, *weights)


def _edge_grid_call(body, arrs, weights):
    blk = lambda w: pl.BlockSpec((_EB, w), lambda i: (i, 0))
    return pl.pallas_call(
        body,
        grid=(_EP // _EB,),
        in_specs=[blk(a.shape[1]) for a in arrs] + [_full(w.shape) for w in weights],
        out_specs=blk(_W),
        out_shape=jax.ShapeDtypeStruct((_EP, _W), jnp.float32),
    )(*arrs, *weights)


def kernel(node_features, edge_features, edge_index,
           enc_n_W1, enc_n_b1, enc_n_W2, enc_n_b2, enc_n_g, enc_n_be,
           enc_e_W1, enc_e_b1, enc_e_W2, enc_e_b2, enc_e_g, enc_e_be,
           pe_W1, pe_b1, pe_W2, pe_b2, pe_g, pe_be,
           pn_W1, pn_b1, pn_W2, pn_b2, pn_g, pn_be,
           dec_W1, dec_b1, dec_W2, dec_b2):
    f32 = jnp.float32
    i32 = jnp.int32
    r = lambda b: b.reshape(1, -1)

    src = edge_index[0].astype(i32)
    dst = edge_index[1].astype(i32)
    srcp = jnp.concatenate([src, jnp.zeros((_EP - _E,), i32)]).reshape(_NW * _NCH, _CH)
    dstp = jnp.concatenate([dst, jnp.full((_EP - _E,), _N, i32)]).reshape(_NW * _NCH, _CH)

    nf = jnp.pad(node_features, ((0, 0), (0, 16 - 9)))
    ef = jnp.pad(edge_features, ((0, _EP - _E), (0, 8 - 2)))
    enW1 = jnp.pad(enc_n_W1, ((0, 16 - 9), (0, 0)))
    eeW1 = jnp.pad(enc_e_W1, ((0, 8 - 2), (0, 0)))
    zero_chunk = jnp.zeros((_CH, _W), f32)

    x, w = _node_grid_call(
        _enc_node_body, [nf],
        [enW1, r(enc_n_b1), enc_n_W2, r(enc_n_b2), r(enc_n_g), r(enc_n_be),
         pe_W1[0, _H:2 * _H], pe_W1[0, 2 * _H:]])
    e2 = _edge_grid_call(
        _enc_edge_body, [ef],
        [eeW1, r(enc_e_b1), enc_e_W2, r(enc_e_b2), r(enc_e_g), r(enc_e_be)])

    for i in range(_P):
        ga, gb = _sc_gather2(w, srcp, dstp)
        e2 = _edge_grid_call(
            _edge_step_body, [e2, ga, gb],
            [pe_W1[i, :_H], r(pe_b1[i]), pe_W2[i], r(pe_b2[i]),
             r(pe_g[i]), r(pe_be[i])])
        so = _sc_scatter(e2, dstp, zero_chunk)
        a0 = so[:_N]
        a1 = so[_ACC:_ACC + _N]
        j = min(i + 1, _P - 1)
        x, w = _node_grid_call(
            _node_step_body, [x, a0, a1],
            [pn_W1[i, :_H], pn_W1[i, _H:], r(pn_b1[i]), pn_W2[i], r(pn_b2[i]),
             r(pn_g[i]), r(pn_be[i]), pe_W1[j, _H:2 * _H], pe_W1[j, 2 * _H:]])

    return pl.pallas_call(
        _dec_body,
        grid=(_N // _NB,),
        in_specs=[pl.BlockSpec((_NB, _H), lambda i: (i, 0)),
                  _full(dec_W1.shape), _full((1, _H)),
                  _full(dec_W2.shape), _full((1, 1))],
        out_specs=pl.BlockSpec((_NB, 1), lambda i: (i, 0)),
        out_shape=jax.ShapeDtypeStruct((_N, 1), f32),
    )(x, dec_W1, r(dec_b1), dec_W2, dec_b2.reshape(1, 1))


# speedup vs baseline: 2.5472x; 1.2780x over previous
import functools

import jax
import jax.numpy as jnp
from jax import lax
from jax.experimental import pallas as pl
from jax.experimental.pallas import tpu as pltpu
from jax.experimental.pallas import tpu_sc as plsc

_N = 10000
_NP = _N // 2
_E = 160000
_H = 64
_W = 128
_P = 8

_NC = 2
_NS = 16
_NW = _NC * _NS
_CH = 128
_NCH = 40
_EPW = _CH * _NCH
_EP = _EPW * _NW
_ACC = 6144
_RPT = _ACC // _NS
_DUMMY = _NP

_EB = 4096
_NBP = 1000

_sc_mesh = plsc.VectorSubcoreMesh(core_axis_name="c", subcore_axis_name="s")


@functools.partial(
    pl.kernel,
    mesh=_sc_mesh,
    out_type=[
        jax.ShapeDtypeStruct((_EP, _W), jnp.float32),
        jax.ShapeDtypeStruct((_EP, _W), jnp.float32),
    ],
    scratch_types=[
        pltpu.VMEM((_NCH, _CH), jnp.int32),
        pltpu.VMEM((_NCH, _CH), jnp.int32),
        pltpu.VMEM((_CH, _W), jnp.float32),
        pltpu.VMEM((_CH, _W), jnp.float32),
        pltpu.VMEM_SHARED((_NP, _W), jnp.float32),
        pltpu.SemaphoreType.DMA,
        pltpu.SemaphoreType.DMA,
    ],
)
def _sc_gather2(up_hbm, vp_hbm, src2_hbm, dst2_hbm, ga_hbm, gb_hbm,
                idxs, idxd, b0, b1, tbl, s0, s1):
    wid = lax.axis_index("s") * _NC + lax.axis_index("c")
    sid = lax.axis_index("s")
    crow = wid * _NCH
    erow = wid * _EPW
    nrt = 320
    row0 = pl.multiple_of(jnp.minimum(sid * nrt, _NP - nrt), 8)

    def run_phase(tbl_hbm, idx_hbm, idxv, out_hbm):
        pltpu.sync_copy(tbl_hbm.at[pl.ds(row0, nrt)], tbl.at[pl.ds(row0, nrt)])
        pltpu.sync_copy(idx_hbm.at[pl.ds(crow, _NCH)], idxv)
        plsc.subcore_barrier()

        def start(j):
            p = j % 2
            return pltpu.async_copy(tbl.at[idxv.at[j]], (b0, b1)[p], (s0, s1)[p])

        pend = [start(0), start(1)]
        for j in range(_NCH):
            p = j % 2
            pend[p].wait()
            pltpu.sync_copy((b0, b1)[p], out_hbm.at[pl.ds(erow + j * _CH, _CH)])
            if j + 2 < _NCH:
                pend[p] = start(j + 2)
        plsc.subcore_barrier()

    run_phase(up_hbm, src2_hbm, idxs, ga_hbm)
    run_phase(vp_hbm, dst2_hbm, idxd, gb_hbm)


@functools.partial(
    pl.kernel,
    mesh=_sc_mesh,
    out_type=jax.ShapeDtypeStruct((_NC * _ACC, _W), jnp.float32),
    scratch_types=[
        pltpu.VMEM((_NCH, _CH), jnp.int32),
        pltpu.VMEM((_CH, _W), jnp.float32),
        pltpu.VMEM((_CH, _W), jnp.float32),
        pltpu.VMEM_SHARED((_ACC, _W), jnp.float32),
        pltpu.SemaphoreType.DMA,
        pltpu.SemaphoreType.DMA,
    ],
)
def _sc_scatter(vals_hbm, dsti_hbm, zero_hbm, out_hbm,
                idx_v, buf0, buf1, acc, sem0, sem1):
    cid = lax.axis_index("c")
    sid = lax.axis_index("s")
    wid = sid * _NC + cid

    pltpu.sync_copy(zero_hbm, buf0)
    for k in range(_RPT // _CH):
        pltpu.sync_copy(buf0, acc.at[pl.ds(sid * _RPT + k * _CH, _CH)])

    pltpu.sync_copy(dsti_hbm.at[pl.ds(wid * _NCH, _NCH)], idx_v)

    def start(j):
        p = j % 2
        return pltpu.async_copy(
            vals_hbm.at[pl.ds(wid * _EPW + j * _CH, _CH)],
            (buf0, buf1)[p], (sem0, sem1)[p])

    plsc.subcore_barrier()
    pend = [start(0), start(1)]
    for j in range(_NCH):
        p = j % 2
        pend[p].wait()
        pltpu.sync_copy((buf0, buf1)[p], acc.at[idx_v.at[j]], add=True)
        if j + 2 < _NCH:
            pend[p] = start(j + 2)
    plsc.subcore_barrier()

    pltpu.sync_copy(acc.at[pl.ds(sid * _RPT, _RPT)],
                    out_hbm.at[pl.ds(cid * _ACC + sid * _RPT, _RPT)])


def _dot(a, b):
    return jnp.dot(a, b, preferred_element_type=jnp.float32)


def _full(shape):
    return pl.BlockSpec(shape, lambda i: tuple(0 for _ in shape))


def _ln(h, g, be):
    mu = jnp.mean(h, axis=-1, keepdims=True)
    var = jnp.mean((h - mu) ** 2, axis=-1, keepdims=True)
    return g * (h - mu) * lax.rsqrt(var + 1e-5) + be


def _ln_packed(h, Mavg, g, be):
    mu = _dot(h, Mavg)
    d = h - mu
    var = _dot(d * d, Mavg)
    return g * d * lax.rsqrt(var + 1e-5) + be


def _enc_node_body(nfp, W1, b1, W2, b2, g, be, Mavg, Ws2, Wd2,
                   x_out, u_out, v_out):
    h = jnp.maximum(_dot(nfp[...], W1[...]) + b1[...], 0.0)
    h = _dot(h, W2[...]) + b2[...]
    xp = _ln_packed(h, Mavg[...], g[...], be[...])
    x_out[...] = xp
    u_out[...] = _dot(xp, Ws2[...])
    v_out[...] = _dot(xp, Wd2[...])


def _enc_edge_body(ef, pd, W1, b1, W2, b2, g, be, e_out):
    h = jnp.maximum(_dot(ef[...], W1[...]) + b1[...], 0.0)
    h = _dot(h, W2[...]) + b2[...]
    en = _ln(h, g[...], be[...])
    p = pd[...]
    e_out[...] = jnp.concatenate([en * (1.0 - p), en * p], axis=1)


def _edge_step_body(e2, ga, gb, ps, pd, W1e, b1, W2, b2, g, be, e_out):
    el = e2[...][:, :_H] + e2[...][:, _H:]
    qs = ps[...]
    qd = pd[...]
    s = (ga[...][:, :_H] * (1.0 - qs) + ga[...][:, _H:] * qs
         + gb[...][:, :_H] * (1.0 - qd) + gb[...][:, _H:] * qd)
    h = jnp.maximum(_dot(el, W1e[...]) + s + b1[...], 0.0)
    h = _dot(h, W2[...]) + b2[...]
    en = _ln(h, g[...], be[...]) + el
    e_out[...] = jnp.concatenate([en * (1.0 - qd), en * qd], axis=1)


def _node_step_body(xp, a0, a1, W1x, W1a, b1, W2, b2, g, be, Mavg, Ws2, Wd2,
                    x_out, u_out, v_out):
    aggp = a0[...] + a1[...]
    h = jnp.maximum(_dot(xp[...], W1x[...]) + _dot(aggp, W1a[...]) + b1[...],
                    0.0)
    h = _dot(h, W2[...]) + b2[...]
    xn = _ln_packed(h, Mavg[...], g[...], be[...]) + xp[...]
    x_out[...] = xn
    u_out[...] = _dot(xn, Ws2[...])
    v_out[...] = _dot(xn, Wd2[...])


def _dec_body(xp, W1, b1, W2, b2, y_out):
    h = jnp.maximum(_dot(xp[...], W1[...]) + b1[...], 0.0)
    y_out[...] = _dot(h, W2[...]) + b2[...]


def _node_grid_call(body, arrs, weights):
    blk = lambda w: pl.BlockSpec((_NBP, w), lambda i: (i, 0))
    f32 = jnp.float32
    return pl.pallas_call(
        body,
        grid=(_NP // _NBP,),
        in_specs=[blk(a.shape[1]) for a in arrs] + [_full(w.shape) for w in weights],
        out_specs=[blk(_W)] * 3,
        out_shape=[jax.ShapeDtypeStruct((_NP, _W), f32)] * 3,
    )(*arrs, *weights)


def _edge_grid_call(body, arrs, weights):
    blk = lambda w: pl.BlockSpec((_EB, w), lambda i: (i, 0))
    return pl.pallas_call(
        body,
        grid=(_EP // _EB,),
        in_specs=[blk(a.shape[1]) for a in arrs] + [_full(w.shape) for w in weights],
        out_specs=blk(_W),
        out_shape=jax.ShapeDtypeStruct((_EP, _W), jnp.float32),
    )(*arrs, *weights)


def _bd(A, B):
    ra, ca = A.shape
    rb, cb = B.shape
    z = jnp.zeros
    return jnp.concatenate([
        jnp.concatenate([A, z((ra, cb), A.dtype)], axis=1),
        jnp.concatenate([z((rb, ca), B.dtype), B], axis=1)], axis=0)


def _row2(b):
    return jnp.concatenate([b, b]).reshape(1, -1)


def kernel(node_features, edge_features, edge_index,
           enc_n_W1, enc_n_b1, enc_n_W2, enc_n_b2, enc_n_g, enc_n_be,
           enc_e_W1, enc_e_b1, enc_e_W2, enc_e_b2, enc_e_g, enc_e_be,
           pe_W1, pe_b1, pe_W2, pe_b2, pe_g, pe_be,
           pn_W1, pn_b1, pn_W2, pn_b2, pn_g, pn_be,
           dec_W1, dec_b1, dec_W2, dec_b2):
    f32 = jnp.float32
    i32 = jnp.int32
    r = lambda b: b.reshape(1, -1)

    src = edge_index[0].astype(i32)
    dst = edge_index[1].astype(i32)
    zpad = jnp.zeros((_EP - _E,), i32)
    src2p = jnp.concatenate([src >> 1, zpad]).reshape(_NW * _NCH, _CH)
    dst2g = jnp.concatenate([dst >> 1, zpad]).reshape(_NW * _NCH, _CH)
    dst2p = jnp.concatenate([dst >> 1, jnp.full((_EP - _E,), _DUMMY, i32)]
                            ).reshape(_NW * _NCH, _CH)
    fpad = jnp.zeros((_EP - _E,), f32)
    ps = jnp.concatenate([(src & 1).astype(f32), fpad]).reshape(_EP, 1)
    pd = jnp.concatenate([(dst & 1).astype(f32), fpad]).reshape(_EP, 1)

    nfp = jnp.pad(node_features, ((0, 0), (0, 16 - 9))).reshape(_NP, 32)
    ef = jnp.pad(edge_features, ((0, _EP - _E), (0, 8 - 2)))
    zero_chunk = jnp.zeros((_CH, _W), f32)

    Mavg = _bd(jnp.full((_H, _H), 1.0 / _H, f32),
               jnp.full((_H, _H), 1.0 / _H, f32))

    def proj_weights(i):
        return (_bd(pe_W1[i, _H:2 * _H], pe_W1[i, _H:2 * _H]),
                _bd(pe_W1[i, 2 * _H:], pe_W1[i, 2 * _H:]))

    Ws2, Wd2 = proj_weights(0)
    enW1 = jnp.pad(enc_n_W1, ((0, 16 - 9), (0, 0)))
    xp, up, vp = _node_grid_call(
        _enc_node_body, [nfp],
        [_bd(enW1, enW1), _row2(enc_n_b1), _bd(enc_n_W2, enc_n_W2),
         _row2(enc_n_b2), _row2(enc_n_g), _row2(enc_n_be), Mavg, Ws2, Wd2])

    eeW1 = jnp.pad(enc_e_W1, ((0, 8 - 2), (0, 0)))
    e2 = _edge_grid_call(
        _enc_edge_body, [ef, pd],
        [eeW1, r(enc_e_b1), enc_e_W2, r(enc_e_b2), r(enc_e_g), r(enc_e_be)])

    for i in range(_P):
        ga, gb = _sc_gather2(up, vp, src2p, dst2g)
        e2 = _edge_grid_call(
            _edge_step_body, [e2, ga, gb, ps, pd],
            [pe_W1[i, :_H], r(pe_b1[i]), pe_W2[i], r(pe_b2[i]),
             r(pe_g[i]), r(pe_be[i])])
        so = _sc_scatter(e2, dst2p, zero_chunk)
        a0 = so[:_NP]
        a1 = so[_ACC:_ACC + _NP]
        Ws2, Wd2 = proj_weights(min(i + 1, _P - 1))
        xp, up, vp = _node_grid_call(
            _node_step_body, [xp, a0, a1],
            [_bd(pn_W1[i, :_H], pn_W1[i, :_H]),
             _bd(pn_W1[i, _H:], pn_W1[i, _H:]),
             _row2(pn_b1[i]), _bd(pn_W2[i], pn_W2[i]), _row2(pn_b2[i]),
             _row2(pn_g[i]), _row2(pn_be[i]), Mavg, Ws2, Wd2])

    yp = pl.pallas_call(
        _dec_body,
        grid=(_NP // _NBP,),
        in_specs=[pl.BlockSpec((_NBP, _W), lambda i: (i, 0)),
                  _full((_W, _W)), _full((1, _W)),
                  _full((_W, 2)), _full((1, 2))],
        out_specs=pl.BlockSpec((_NBP, 2), lambda i: (i, 0)),
        out_shape=jax.ShapeDtypeStruct((_NP, 2), f32),
    )(xp, _bd(dec_W1, dec_W1), _row2(dec_b1), _bd(dec_W2, dec_W2),
      _row2(dec_b2))
    return yp.reshape(_N, 1)


# speedup vs baseline: 2.6720x; 1.0490x over previous
import functools

import jax
import jax.numpy as jnp
from jax import lax
from jax.experimental import pallas as pl
from jax.experimental.pallas import tpu as pltpu
from jax.experimental.pallas import tpu_sc as plsc

_N = 10000
_NP = _N // 2
_E = 160000
_H = 64
_W = 128
_P = 8

_NC = 2
_NS = 16
_NW = _NC * _NS
_CH = 128
_NCH = 40
_EPW = _CH * _NCH
_EP = _EPW * _NW
_ACC = 6144
_RPT = _ACC // _NS
_DUMMY = _NP

_EB = 4096
_NBP = 1000

_sc_mesh = plsc.VectorSubcoreMesh(core_axis_name="c", subcore_axis_name="s")


@functools.partial(
    pl.kernel,
    mesh=_sc_mesh,
    out_type=[
        jax.ShapeDtypeStruct((_EP, _W), jnp.float32),
        jax.ShapeDtypeStruct((_EP, _W), jnp.float32),
    ],
    scratch_types=[
        pltpu.VMEM((_NCH, _CH), jnp.int32),
        pltpu.VMEM((_NCH, _CH), jnp.int32),
        pltpu.VMEM((_CH, _W), jnp.float32),
        pltpu.VMEM((_CH, _W), jnp.float32),
        pltpu.VMEM((_CH, _W), jnp.float32),
        pltpu.VMEM((_CH, _W), jnp.float32),
        pltpu.VMEM_SHARED((_NP, _W), jnp.float32),
        pltpu.SemaphoreType.DMA,
        pltpu.SemaphoreType.DMA,
        pltpu.SemaphoreType.DMA,
        pltpu.SemaphoreType.DMA,
    ],
)
def _sc_gather2(xp_hbm, src2_hbm, dst2_hbm, ga_hbm, gb_hbm,
                idxs, idxd, b0, b1, b2, b3, tbl, s0, s1, s2, s3):
    wid = lax.axis_index("s") * _NC + lax.axis_index("c")
    sid = lax.axis_index("s")
    crow = wid * _NCH
    erow = wid * _EPW
    nrt = 320
    row0 = pl.multiple_of(jnp.minimum(sid * nrt, _NP - nrt), 8)
    pltpu.sync_copy(xp_hbm.at[pl.ds(row0, nrt)], tbl.at[pl.ds(row0, nrt)])
    pltpu.sync_copy(src2_hbm.at[pl.ds(crow, _NCH)], idxs)
    pltpu.sync_copy(dst2_hbm.at[pl.ds(crow, _NCH)], idxd)
    plsc.subcore_barrier()

    sems = (s0, s1, s2, s3)
    bufs = (b0, b1, b2, b3)

    def start(j):
        idxv = idxs if j % 2 == 0 else idxd
        return pltpu.async_copy(tbl.at[idxv.at[j // 2]], bufs[j % 4],
                                sems[j % 4])

    nj = 2 * _NCH
    pend = [start(j) for j in range(4)]
    for j in range(nj):
        p = j % 4
        pend[p].wait()
        out = ga_hbm if j % 2 == 0 else gb_hbm
        pltpu.sync_copy(bufs[p], out.at[pl.ds(erow + (j // 2) * _CH, _CH)])
        if j + 4 < nj:
            pend[p] = start(j + 4)


@functools.partial(
    pl.kernel,
    mesh=_sc_mesh,
    out_type=jax.ShapeDtypeStruct((_NC * _ACC, _W), jnp.float32),
    scratch_types=[
        pltpu.VMEM((_NCH, _CH), jnp.int32),
        pltpu.VMEM((_CH, _W), jnp.float32),
        pltpu.VMEM((_CH, _W), jnp.float32),
        pltpu.VMEM_SHARED((_ACC, _W), jnp.float32),
        pltpu.SemaphoreType.DMA,
        pltpu.SemaphoreType.DMA,
    ],
)
def _sc_scatter(vals_hbm, dsti_hbm, zero_hbm, out_hbm,
                idx_v, buf0, buf1, acc, sem0, sem1):
    cid = lax.axis_index("c")
    sid = lax.axis_index("s")
    wid = sid * _NC + cid

    pltpu.sync_copy(zero_hbm, buf0)
    for k in range(_RPT // _CH):
        pltpu.sync_copy(buf0, acc.at[pl.ds(sid * _RPT + k * _CH, _CH)])

    pltpu.sync_copy(dsti_hbm.at[pl.ds(wid * _NCH, _NCH)], idx_v)

    def start(j):
        p = j % 2
        return pltpu.async_copy(
            vals_hbm.at[pl.ds(wid * _EPW + j * _CH, _CH)],
            (buf0, buf1)[p], (sem0, sem1)[p])

    plsc.subcore_barrier()
    pend = [start(0), start(1)]
    for j in range(_NCH):
        p = j % 2
        pend[p].wait()
        pltpu.sync_copy((buf0, buf1)[p], acc.at[idx_v.at[j]], add=True)
        if j + 2 < _NCH:
            pend[p] = start(j + 2)
    plsc.subcore_barrier()

    pltpu.sync_copy(acc.at[pl.ds(sid * _RPT, _RPT)],
                    out_hbm.at[pl.ds(cid * _ACC + sid * _RPT, _RPT)])


def _dot(a, b):
    return jnp.dot(a, b, preferred_element_type=jnp.float32)


def _full(shape):
    return pl.BlockSpec(shape, lambda i: tuple(0 for _ in shape))


def _ln(h, g, be):
    mu = jnp.mean(h, axis=-1, keepdims=True)
    var = jnp.mean((h - mu) ** 2, axis=-1, keepdims=True)
    return g * (h - mu) * lax.rsqrt(var + 1e-5) + be


def _ln_packed(h, Mavg, g, be):
    mu = _dot(h, Mavg)
    d = h - mu
    var = _dot(d * d, Mavg)
    return g * d * lax.rsqrt(var + 1e-5) + be


def _enc_node_body(nfp, W1, b1, W2, b2, g, be, Mavg, x_out):
    h = jnp.maximum(_dot(nfp[...], W1[...]) + b1[...], 0.0)
    h = _dot(h, W2[...]) + b2[...]
    x_out[...] = _ln_packed(h, Mavg[...], g[...], be[...])


def _enc_edge_body(ef, pd, W1, b1, W2, b2, g, be, e_out):
    h = jnp.maximum(_dot(ef[...], W1[...]) + b1[...], 0.0)
    h = _dot(h, W2[...]) + b2[...]
    en = _ln(h, g[...], be[...])
    p = pd[...]
    e_out[...] = jnp.concatenate([en * (1.0 - p), en * p], axis=1)


def _edge_step_body(e2, ga, gb, ps, pd, W1e, W1s, W1d, b1, W2, b2, g, be,
                    e_out):
    el = e2[...][:, :_H] + e2[...][:, _H:]
    qs = ps[...]
    qd = pd[...]
    xs = ga[...][:, :_H] * (1.0 - qs) + ga[...][:, _H:] * qs
    xd = gb[...][:, :_H] * (1.0 - qd) + gb[...][:, _H:] * qd
    h = jnp.maximum(_dot(el, W1e[...]) + _dot(xs, W1s[...])
                    + _dot(xd, W1d[...]) + b1[...], 0.0)
    h = _dot(h, W2[...]) + b2[...]
    en = _ln(h, g[...], be[...]) + el
    e_out[...] = jnp.concatenate([en * (1.0 - qd), en * qd], axis=1)


def _node_step_body(xp, a0, a1, W1x, W1a, b1, W2, b2, g, be, Mavg, x_out):
    aggp = a0[...] + a1[...]
    h = jnp.maximum(_dot(xp[...], W1x[...]) + _dot(aggp, W1a[...]) + b1[...],
                    0.0)
    h = _dot(h, W2[...]) + b2[...]
    x_out[...] = _ln_packed(h, Mavg[...], g[...], be[...]) + xp[...]


def _dec_body(xp, W1, b1, W2, b2, y_out):
    h = jnp.maximum(_dot(xp[...], W1[...]) + b1[...], 0.0)
    y_out[...] = _dot(h, W2[...]) + b2[...]


def _node_grid_call(body, arrs, weights):
    blk = lambda w: pl.BlockSpec((_NBP, w), lambda i: (i, 0))
    f32 = jnp.float32
    return pl.pallas_call(
        body,
        grid=(_NP // _NBP,),
        in_specs=[blk(a.shape[1]) for a in arrs] + [_full(w.shape) for w in weights],
        out_specs=blk(_W),
        out_shape=jax.ShapeDtypeStruct((_NP, _W), f32),
    )(*arrs, *weights)


def _edge_grid_call(body, arrs, weights):
    blk = lambda w: pl.BlockSpec((_EB, w), lambda i: (i, 0))
    return pl.pallas_call(
        body,
        grid=(_EP // _EB,),
        in_specs=[blk(a.shape[1]) for a in arrs] + [_full(w.shape) for w in weights],
        out_specs=blk(_W),
        out_shape=jax.ShapeDtypeStruct((_EP, _W), jnp.float32),
    )(*arrs, *weights)


def _bd(A, B):
    ra, ca = A.shape
    rb, cb = B.shape
    z = jnp.zeros
    return jnp.concatenate([
        jnp.concatenate([A, z((ra, cb), A.dtype)], axis=1),
        jnp.concatenate([z((rb, ca), B.dtype), B], axis=1)], axis=0)


def _row2(b):
    return jnp.concatenate([b, b]).reshape(1, -1)


def kernel(node_features, edge_features, edge_index,
           enc_n_W1, enc_n_b1, enc_n_W2, enc_n_b2, enc_n_g, enc_n_be,
           enc_e_W1, enc_e_b1, enc_e_W2, enc_e_b2, enc_e_g, enc_e_be,
           pe_W1, pe_b1, pe_W2, pe_b2, pe_g, pe_be,
           pn_W1, pn_b1, pn_W2, pn_b2, pn_g, pn_be,
           dec_W1, dec_b1, dec_W2, dec_b2):
    f32 = jnp.float32
    i32 = jnp.int32
    r = lambda b: b.reshape(1, -1)

    src = edge_index[0].astype(i32)
    dst = edge_index[1].astype(i32)
    zpad = jnp.zeros((_EP - _E,), i32)
    src2p = jnp.concatenate([src >> 1, zpad]).reshape(_NW * _NCH, _CH)
    dst2g = jnp.concatenate([dst >> 1, zpad]).reshape(_NW * _NCH, _CH)
    dst2p = jnp.concatenate([dst >> 1, jnp.full((_EP - _E,), _DUMMY, i32)]
                            ).reshape(_NW * _NCH, _CH)
    fpad = jnp.zeros((_EP - _E,), f32)
    ps = jnp.concatenate([(src & 1).astype(f32), fpad]).reshape(_EP, 1)
    pd = jnp.concatenate([(dst & 1).astype(f32), fpad]).reshape(_EP, 1)

    nfp = jnp.pad(node_features, ((0, 0), (0, 16 - 9))).reshape(_NP, 32)
    ef = jnp.pad(edge_features, ((0, _EP - _E), (0, 8 - 2)))
    zero_chunk = jnp.zeros((_CH, _W), f32)

    Mavg = _bd(jnp.full((_H, _H), 1.0 / _H, f32),
               jnp.full((_H, _H), 1.0 / _H, f32))

    enW1 = jnp.pad(enc_n_W1, ((0, 16 - 9), (0, 0)))
    xp = _node_grid_call(
        _enc_node_body, [nfp],
        [_bd(enW1, enW1), _row2(enc_n_b1), _bd(enc_n_W2, enc_n_W2),
         _row2(enc_n_b2), _row2(enc_n_g), _row2(enc_n_be), Mavg])

    eeW1 = jnp.pad(enc_e_W1, ((0, 8 - 2), (0, 0)))
    e2 = _edge_grid_call(
        _enc_edge_body, [ef, pd],
        [eeW1, r(enc_e_b1), enc_e_W2, r(enc_e_b2), r(enc_e_g), r(enc_e_be)])

    for i in range(_P):
        ga, gb = _sc_gather2(xp, src2p, dst2g)
        e2 = _edge_grid_call(
            _edge_step_body, [e2, ga, gb, ps, pd],
            [pe_W1[i, :_H], pe_W1[i, _H:2 * _H], pe_W1[i, 2 * _H:],
             r(pe_b1[i]), pe_W2[i], r(pe_b2[i]),
             r(pe_g[i]), r(pe_be[i])])
        so = _sc_scatter(e2, dst2p, zero_chunk)
        a0 = so[:_NP]
        a1 = so[_ACC:_ACC + _NP]
        xp = _node_grid_call(
            _node_step_body, [xp, a0, a1],
            [_bd(pn_W1[i, :_H], pn_W1[i, :_H]),
             _bd(pn_W1[i, _H:], pn_W1[i, _H:]),
             _row2(pn_b1[i]), _bd(pn_W2[i], pn_W2[i]), _row2(pn_b2[i]),
             _row2(pn_g[i]), _row2(pn_be[i]), Mavg])

    yp = pl.pallas_call(
        _dec_body,
        grid=(_NP // _NBP,),
        in_specs=[pl.BlockSpec((_NBP, _W), lambda i: (i, 0)),
                  _full((_W, _W)), _full((1, _W)),
                  _full((_W, 2)), _full((1, 2))],
        out_specs=pl.BlockSpec((_NBP, 2), lambda i: (i, 0)),
        out_shape=jax.ShapeDtypeStruct((_NP, 2), f32),
    )(xp, _bd(dec_W1, dec_W1), _row2(dec_b1), _bd(dec_W2, dec_W2),
      _row2(dec_b2))
    return yp.reshape(_N, 1)


# speedup vs baseline: 2.7406x; 1.0257x over previous
import functools

import jax
import jax.numpy as jnp
from jax import lax
from jax.experimental import pallas as pl
from jax.experimental.pallas import tpu as pltpu
from jax.experimental.pallas import tpu_sc as plsc

_N = 10000
_NP = _N // 2
_E = 160000
_EH = _E // 2
_H = 64
_W = 128
_P = 8

_NC = 2
_NS = 16
_NW = _NC * _NS
_CH = 128
_NCH = 20
_NCHP = 24
_EPW = _CH * _NCH
_EP = _EPW * _NW
_ACC = 6144
_RPT = _ACC // _NS
_DUMMY = _NP

_EB = 4096
_NBP = 1000

_sc_mesh = plsc.VectorSubcoreMesh(core_axis_name="c", subcore_axis_name="s")


@functools.partial(
    pl.kernel,
    mesh=_sc_mesh,
    out_type=[
        jax.ShapeDtypeStruct((_EP, _W), jnp.float32),
        jax.ShapeDtypeStruct((_EP, _W), jnp.float32),
    ],
    scratch_types=[
        pltpu.VMEM((_NCHP, _CH), jnp.int32),
        pltpu.VMEM((_NCHP, _CH), jnp.int32),
        pltpu.VMEM((_CH, _W), jnp.float32),
        pltpu.VMEM((_CH, _W), jnp.float32),
        pltpu.VMEM((_CH, _W), jnp.float32),
        pltpu.VMEM((_CH, _W), jnp.float32),
        pltpu.VMEM_SHARED((_NP, _W), jnp.float32),
        pltpu.SemaphoreType.DMA,
        pltpu.SemaphoreType.DMA,
        pltpu.SemaphoreType.DMA,
        pltpu.SemaphoreType.DMA,
    ],
)
def _sc_gather2(xp_hbm, src2_hbm, dst2_hbm, ga_hbm, gb_hbm,
                idxs, idxd, b0, b1, b2, b3, tbl, s0, s1, s2, s3):
    wid = lax.axis_index("s") * _NC + lax.axis_index("c")
    sid = lax.axis_index("s")
    crow = pl.multiple_of(wid * _NCHP, 8)
    erow = wid * _EPW
    nrt = 320
    row0 = pl.multiple_of(jnp.minimum(sid * nrt, _NP - nrt), 8)
    pltpu.sync_copy(xp_hbm.at[pl.ds(row0, nrt)], tbl.at[pl.ds(row0, nrt)])
    pltpu.sync_copy(src2_hbm.at[pl.ds(crow, _NCHP)], idxs)
    pltpu.sync_copy(dst2_hbm.at[pl.ds(crow, _NCHP)], idxd)
    plsc.subcore_barrier()

    sems = (s0, s1, s2, s3)
    bufs = (b0, b1, b2, b3)

    def start(j):
        idxv = idxs if j % 2 == 0 else idxd
        return pltpu.async_copy(tbl.at[idxv.at[j // 2]], bufs[j % 4],
                                sems[j % 4])

    nj = 2 * _NCH
    pend = [start(j) for j in range(4)]
    for j in range(nj):
        p = j % 4
        pend[p].wait()
        out = ga_hbm if j % 2 == 0 else gb_hbm
        pltpu.sync_copy(bufs[p], out.at[pl.ds(erow + (j // 2) * _CH, _CH)])
        if j + 4 < nj:
            pend[p] = start(j + 4)


@functools.partial(
    pl.kernel,
    mesh=_sc_mesh,
    out_type=jax.ShapeDtypeStruct((_NC * _ACC, _W), jnp.float32),
    scratch_types=[
        pltpu.VMEM((_NCHP, _CH), jnp.int32),
        pltpu.VMEM((_CH, _W), jnp.float32),
        pltpu.VMEM((_CH, _W), jnp.float32),
        pltpu.VMEM_SHARED((_ACC, _W), jnp.float32),
        pltpu.SemaphoreType.DMA,
        pltpu.SemaphoreType.DMA,
    ],
)
def _sc_scatter(vals_hbm, dsti_hbm, zero_hbm, out_hbm,
                idx_v, buf0, buf1, acc, sem0, sem1):
    cid = lax.axis_index("c")
    sid = lax.axis_index("s")
    wid = sid * _NC + cid

    pltpu.sync_copy(zero_hbm, buf0)
    for k in range(_RPT // _CH):
        pltpu.sync_copy(buf0, acc.at[pl.ds(sid * _RPT + k * _CH, _CH)])

    pltpu.sync_copy(dsti_hbm.at[pl.ds(pl.multiple_of(wid * _NCHP, 8), _NCHP)],
                    idx_v)

    def start(j):
        p = j % 2
        return pltpu.async_copy(
            vals_hbm.at[pl.ds(wid * _EPW + j * _CH, _CH)],
            (buf0, buf1)[p], (sem0, sem1)[p])

    plsc.subcore_barrier()
    pend = [start(0), start(1)]
    for j in range(_NCH):
        p = j % 2
        pend[p].wait()
        pltpu.sync_copy((buf0, buf1)[p], acc.at[idx_v.at[j]], add=True)
        if j + 2 < _NCH:
            pend[p] = start(j + 2)
    plsc.subcore_barrier()

    pltpu.sync_copy(acc.at[pl.ds(sid * _RPT, _RPT)],
                    out_hbm.at[pl.ds(cid * _ACC + sid * _RPT, _RPT)])


def _dot(a, b):
    return jnp.dot(a, b, preferred_element_type=jnp.float32)


def _full(shape):
    return pl.BlockSpec(shape, lambda i: tuple(0 for _ in shape))


def _ln(h, g, be):
    mu = jnp.mean(h, axis=-1, keepdims=True)
    var = jnp.mean((h - mu) ** 2, axis=-1, keepdims=True)
    return g * (h - mu) * lax.rsqrt(var + 1e-5) + be


def _ln_packed(h, Mavg, g, be):
    mu = _dot(h, Mavg)
    d = h - mu
    var = _dot(d * d, Mavg)
    return g * d * lax.rsqrt(var + 1e-5) + be


def _enc_node_body(nfp, W1, b1, W2, b2, g, be, Mavg, x_out):
    h = jnp.maximum(_dot(nfp[...], W1[...]) + b1[...], 0.0)
    h = _dot(h, W2[...]) + b2[...]
    x_out[...] = _ln_packed(h, Mavg[...], g[...], be[...])


def _enc_edge_body(ef, pd, W1, b1, W2, b2, g, be, e_out):
    h = jnp.maximum(_dot(ef[...], W1[...]) + b1[...], 0.0)
    h = _dot(h, W2[...]) + b2[...]
    en = _ln(h, g[...], be[...])
    p = pd[...]
    e_out[...] = jnp.concatenate([en * (1.0 - p), en * p], axis=1)


def _edge_step_body(e2, ga, gb, ps, pd, W1e, W1s, W1d, b1, W2, b2, g, be,
                    e_out):
    el = e2[...][:, :_H] + e2[...][:, _H:]
    qs = ps[...]
    qd = pd[...]
    xs = ga[...][:, :_H] * (1.0 - qs) + ga[...][:, _H:] * qs
    xd = gb[...][:, :_H] * (1.0 - qd) + gb[...][:, _H:] * qd
    h = jnp.maximum(_dot(el, W1e[...]) + _dot(xs, W1s[...])
                    + _dot(xd, W1d[...]) + b1[...], 0.0)
    h = _dot(h, W2[...]) + b2[...]
    en = _ln(h, g[...], be[...]) + el
    e_out[...] = jnp.concatenate([en * (1.0 - qd), en * qd], axis=1)


def _node_step_body(xp, a0A, a1A, a0B, a1B, W1x, W1a, b1, W2, b2, g, be, Mavg,
                    x_out):
    aggp = (a0A[...] + a1A[...]) + (a0B[...] + a1B[...])
    h = jnp.maximum(_dot(xp[...], W1x[...]) + _dot(aggp, W1a[...]) + b1[...],
                    0.0)
    h = _dot(h, W2[...]) + b2[...]
    x_out[...] = _ln_packed(h, Mavg[...], g[...], be[...]) + xp[...]


def _dec_body(xp, W1, b1, W2, b2, y_out):
    h = jnp.maximum(_dot(xp[...], W1[...]) + b1[...], 0.0)
    y_out[...] = _dot(h, W2[...]) + b2[...]


def _node_grid_call(body, arrs, weights):
    blk = lambda w: pl.BlockSpec((_NBP, w), lambda i: (i, 0))
    f32 = jnp.float32
    return pl.pallas_call(
        body,
        grid=(_NP // _NBP,),
        in_specs=[blk(a.shape[1]) for a in arrs] + [_full(w.shape) for w in weights],
        out_specs=blk(_W),
        out_shape=jax.ShapeDtypeStruct((_NP, _W), f32),
    )(*arrs, *weights)


def _edge_grid_call(body, arrs, weights):
    blk = lambda w: pl.BlockSpec((_EB, w), lambda i: (i, 0))
    return pl.pallas_call(
        body,
        grid=(_EP // _EB,),
        in_specs=[blk(a.shape[1]) for a in arrs] + [_full(w.shape) for w in weights],
        out_specs=blk(_W),
        out_shape=jax.ShapeDtypeStruct((_EP, _W), jnp.float32),
    )(*arrs, *weights)


def _bd(A, B):
    ra, ca = A.shape
    rb, cb = B.shape
    z = jnp.zeros
    return jnp.concatenate([
        jnp.concatenate([A, z((ra, cb), A.dtype)], axis=1),
        jnp.concatenate([z((rb, ca), B.dtype), B], axis=1)], axis=0)


def _row2(b):
    return jnp.concatenate([b, b]).reshape(1, -1)


def kernel(node_features, edge_features, edge_index,
           enc_n_W1, enc_n_b1, enc_n_W2, enc_n_b2, enc_n_g, enc_n_be,
           enc_e_W1, enc_e_b1, enc_e_W2, enc_e_b2, enc_e_g, enc_e_be,
           pe_W1, pe_b1, pe_W2, pe_b2, pe_g, pe_be,
           pn_W1, pn_b1, pn_W2, pn_b2, pn_g, pn_be,
           dec_W1, dec_b1, dec_W2, dec_b2):
    f32 = jnp.float32
    i32 = jnp.int32
    r = lambda b: b.reshape(1, -1)

    src = edge_index[0].astype(i32)
    dst = edge_index[1].astype(i32)
    zpad = jnp.zeros((_EP - _EH,), i32)
    dpad = jnp.full((_EP - _EH,), _DUMMY, i32)
    fpad = jnp.zeros((_EP - _EH,), f32)

    def stripe(v):
        return jnp.pad(v.reshape(_NW, _NCH, _CH),
                       ((0, 0), (0, _NCHP - _NCH), (0, 0))
                       ).reshape(_NW * _NCHP, _CH)

    def half_idx(lo):
        s = lax.slice(src, (lo,), (lo + _EH,))
        d = lax.slice(dst, (lo,), (lo + _EH,))
        return (stripe(jnp.concatenate([s >> 1, zpad])),
                stripe(jnp.concatenate([d >> 1, zpad])),
                stripe(jnp.concatenate([d >> 1, dpad])),
                jnp.concatenate([(s & 1).astype(f32), fpad]).reshape(_EP, 1),
                jnp.concatenate([(d & 1).astype(f32), fpad]).reshape(_EP, 1))

    srcA, dstA, dstAp, psA, pdA = half_idx(0)
    srcB, dstB, dstBp, psB, pdB = half_idx(_EH)

    nfp = jnp.pad(node_features, ((0, 0), (0, 16 - 9))).reshape(_NP, 32)
    ef = jnp.pad(edge_features, ((0, 0), (0, 8 - 2)))
    efA = jnp.pad(lax.slice(ef, (0, 0), (_EH, 8)), ((0, _EP - _EH), (0, 0)))
    efB = jnp.pad(lax.slice(ef, (_EH, 0), (_E, 8)), ((0, _EP - _EH), (0, 0)))
    zero_chunk = jnp.zeros((_CH, _W), f32)

    Mavg = _bd(jnp.full((_H, _H), 1.0 / _H, f32),
               jnp.full((_H, _H), 1.0 / _H, f32))

    enW1 = jnp.pad(enc_n_W1, ((0, 16 - 9), (0, 0)))
    xp = _node_grid_call(
        _enc_node_body, [nfp],
        [_bd(enW1, enW1), _row2(enc_n_b1), _bd(enc_n_W2, enc_n_W2),
         _row2(enc_n_b2), _row2(enc_n_g), _row2(enc_n_be), Mavg])

    eeW1 = jnp.pad(enc_e_W1, ((0, 8 - 2), (0, 0)))
    enc_e_w = [eeW1, r(enc_e_b1), enc_e_W2, r(enc_e_b2), r(enc_e_g),
               r(enc_e_be)]
    e2A = _edge_grid_call(_enc_edge_body, [efA, pdA], enc_e_w)
    e2B = _edge_grid_call(_enc_edge_body, [efB, pdB], enc_e_w)

    for i in range(_P):
        ew = [pe_W1[i, :_H], pe_W1[i, _H:2 * _H], pe_W1[i, 2 * _H:],
              r(pe_b1[i]), pe_W2[i], r(pe_b2[i]), r(pe_g[i]), r(pe_be[i])]
        gaA, gbA = _sc_gather2(xp, srcA, dstA)
        gaB, gbB = _sc_gather2(xp, srcB, dstB)
        e2A = _edge_grid_call(_edge_step_body, [e2A, gaA, gbA, psA, pdA], ew)
        soA = _sc_scatter(e2A, dstAp, zero_chunk)
        e2B = _edge_grid_call(_edge_step_body, [e2B, gaB, gbB, psB, pdB], ew)
        soB = _sc_scatter(e2B, dstBp, zero_chunk)
        xp = _node_grid_call(
            _node_step_body,
            [xp, soA[:_NP], soA[_ACC:_ACC + _NP],
             soB[:_NP], soB[_ACC:_ACC + _NP]],
            [_bd(pn_W1[i, :_H], pn_W1[i, :_H]),
             _bd(pn_W1[i, _H:], pn_W1[i, _H:]),
             _row2(pn_b1[i]), _bd(pn_W2[i], pn_W2[i]), _row2(pn_b2[i]),
             _row2(pn_g[i]), _row2(pn_be[i]), Mavg])

    yp = pl.pallas_call(
        _dec_body,
        grid=(_NP // _NBP,),
        in_specs=[pl.BlockSpec((_NBP, _W), lambda i: (i, 0)),
                  _full((_W, _W)), _full((1, _W)),
                  _full((_W, 2)), _full((1, 2))],
        out_specs=pl.BlockSpec((_NBP, 2), lambda i: (i, 0)),
        out_shape=jax.ShapeDtypeStruct((_NP, 2), f32),
    )(xp, _bd(dec_W1, dec_W1), _row2(dec_b1), _bd(dec_W2, dec_W2),
      _row2(dec_b2))
    return yp.reshape(_N, 1)
